# baseline (speedup 1.0000x reference)
import functools

import jax
import jax.numpy as jnp
from jax.experimental import pallas as pl
from jax.experimental.pallas import tpu as pltpu

_LANES = 128
_NEG = -(2**31)
_BIGIDX = 2**31 - 1


def _to_sortable_i32(x):
    s = jax.lax.bitcast_convert_type(x, jnp.int32)
    return s ^ (jax.lax.shift_right_arithmetic(s, 31) & jnp.int32(0x7FFFFFFF))


def _from_sortable_i32(y):
    s = y ^ (jax.lax.shift_right_arithmetic(y, 31) & jnp.int32(0x7FFFFFFF))
    return jax.lax.bitcast_convert_type(s, jnp.float32)


def _topk_body(x_ref, probs_ref, idx_ref, y_ref, *, V, Vp, R, K, RUN):
    S = Vp // _LANES

    x = x_ref[...]
    col = jax.lax.broadcasted_iota(jnp.int32, (R, Vp), 1)
    y = jnp.where(col < V, _to_sortable_i32(x), _NEG)
    y_ref[...] = y

    lane = jax.lax.broadcasted_iota(jnp.int32, (R, _LANES), 1)

    def round_body(carry):
        rv, ri, _ = carry
        yr = y_ref[...].reshape(R, S, _LANES)
        colmax = jnp.max(yr, axis=1)
        si = jax.lax.broadcasted_iota(jnp.int32, (R, S, _LANES), 1)
        hit = yr == colmax[:, None, :]
        smin = jnp.min(jnp.where(hit, si, jnp.int32(S)), axis=1)
        vidx = smin * _LANES + lane
        y_ref[...] = jnp.where(si == smin[:, None, :], _NEG, yr).reshape(R, Vp)
        bankmax = jnp.max(colmax, axis=1)
        done = jnp.all(bankmax < rv[:, K - 1])
        mv = jnp.concatenate([rv, colmax], axis=1)
        mi = jnp.concatenate([ri, vidx], axis=1)
        outs_v = []
        outs_i = []
        for _ in range(K):
            m = jnp.max(mv, axis=1, keepdims=True)
            eq = mv == m
            ci = jnp.min(jnp.where(eq, mi, _BIGIDX), axis=1, keepdims=True)
            outs_v.append(m)
            outs_i.append(ci)
            mv = jnp.where(eq & (mi == ci), _NEG, mv)
        pad_v = jnp.full((R, RUN - K), _NEG, jnp.int32)
        pad_i = jnp.full((R, RUN - K), _BIGIDX, jnp.int32)
        rv2 = jnp.concatenate(outs_v + [pad_v], axis=1)
        ri2 = jnp.concatenate(outs_i + [pad_i], axis=1)
        return rv2, ri2, done

    init = (
        jnp.full((R, RUN), _NEG, jnp.int32),
        jnp.full((R, RUN), _BIGIDX, jnp.int32),
        jnp.bool_(False),
    )
    rv, ri, _ = jax.lax.while_loop(lambda c: jnp.logical_not(c[2]), round_body, init)

    v = _from_sortable_i32(rv[:, :K])
    m = jnp.max(v, axis=1, keepdims=True)
    e = jnp.exp(v - m)
    probs_ref[...] = e / jnp.sum(e, axis=1, keepdims=True)
    idx_ref[...] = ri[:, :K]


def _topk_probs(logits, K, R, interpret=False):
    B, V = logits.shape
    Vp = ((V + _LANES - 1) // _LANES) * _LANES
    RUN = 64
    body = functools.partial(_topk_body, V=V, Vp=Vp, R=R, K=K, RUN=RUN)
    probs, idx = pl.pallas_call(
        body,
        grid=(B // R,),
        in_specs=[pl.BlockSpec((R, Vp), lambda i: (i, 0))],
        out_specs=[
            pl.BlockSpec((R, K), lambda i: (i, 0)),
            pl.BlockSpec((R, K), lambda i: (i, 0)),
        ],
        out_shape=[
            jax.ShapeDtypeStruct((B, K), jnp.float32),
            jax.ShapeDtypeStruct((B, K), jnp.int32),
        ],
        scratch_shapes=[pltpu.VMEM((R, Vp), jnp.int32)],
        interpret=interpret,
    )(logits)
    return probs, idx


@jax.jit
def kernel(logits):
    B, V = logits.shape
    K = min(50, V)
    R = 8 if B % 8 == 0 else 1
    probs, indices = _topk_probs(logits, K, R)
    sample_key = jax.random.key(42)
    sampled = jax.random.categorical(sample_key, jnp.log(probs + 1e-20), axis=-1)
    tokens = jnp.take_along_axis(indices, sampled[:, None], axis=1).squeeze(-1)
    return tokens, probs


# speedup vs baseline: 2.2661x; 2.2661x over previous
import functools

import jax
import jax.numpy as jnp
from jax.experimental import pallas as pl
from jax.experimental.pallas import tpu as pltpu

_LANES = 128
_NEG = -(2**31)
_BIGIDX = 2**31 - 1
_NBANKS = 4


def _to_sortable_i32(x):
    s = jax.lax.bitcast_convert_type(x, jnp.int32)
    return s ^ (jax.lax.shift_right_arithmetic(s, 31) & jnp.int32(0x7FFFFFFF))


def _from_sortable_i32(y):
    s = y ^ (jax.lax.shift_right_arithmetic(y, 31) & jnp.int32(0x7FFFFFFF))
    return jax.lax.bitcast_convert_type(s, jnp.float32)


def _rank_compact(cv, ci, wout):
    R, W = cv.shape
    ranks = jnp.zeros((R, W), jnp.int32)
    CH = 128
    for j0 in range(0, W, CH):
        cvj = jax.lax.broadcast_in_dim(cv[:, j0:j0 + CH], (R, CH, W), (0, 1))
        cij = jax.lax.broadcast_in_dim(ci[:, j0:j0 + CH], (R, CH, W), (0, 1))
        cvi = jax.lax.broadcast_in_dim(cv, (R, CH, W), (0, 2))
        cii = jax.lax.broadcast_in_dim(ci, (R, CH, W), (0, 2))
        better = (cvj > cvi) | ((cvj == cvi) & (cij < cii))
        ranks = ranks + jnp.sum(better.astype(jnp.int32), axis=1)
    slot = jax.lax.broadcasted_iota(jnp.int32, (R, W, wout), 2)
    oh = (ranks[:, :, None] == slot).astype(jnp.float32)
    lo = (cv & jnp.int32(0xFFFF)).astype(jnp.float32)
    hi = jax.lax.shift_right_arithmetic(cv, 16).astype(jnp.float32)
    ilo = (ci & jnp.int32(0xFFFF)).astype(jnp.float32)
    ihi = jax.lax.shift_right_arithmetic(ci, 16).astype(jnp.float32)
    dn = (((1,), (1,)), ((0,), (0,)))
    mm = functools.partial(
        jax.lax.dot_general, dimension_numbers=dn,
        preferred_element_type=jnp.float32,
        precision=jax.lax.Precision.HIGHEST)
    slo = mm(lo, oh).astype(jnp.int32)
    shi = mm(hi, oh).astype(jnp.int32)
    silo = mm(ilo, oh).astype(jnp.int32)
    sihi = mm(ihi, oh).astype(jnp.int32)
    scv = jax.lax.shift_left(shi, 16) | (slo & jnp.int32(0xFFFF))
    sci = jax.lax.shift_left(sihi, 16) | (silo & jnp.int32(0xFFFF))
    return scv, sci


def _topk_body(x_ref, probs_ref, idx_ref, y_ref, *, V, Vp, R, K):
    S = Vp // _LANES
    W = _NBANKS * _LANES

    x = x_ref[...]
    col = jax.lax.broadcasted_iota(jnp.int32, (R, Vp), 1)
    y = jnp.where(col < V, _to_sortable_i32(x), jnp.int32(_NEG))
    y_ref[...] = y

    lane = jax.lax.broadcasted_iota(jnp.int32, (R, _LANES), 1)

    def colmax_of(yr):
        return jnp.max(yr, axis=1)

    def extract_bank(yr, colmax):
        si = jax.lax.broadcasted_iota(jnp.int32, (R, S, _LANES), 1)
        hit = yr == colmax[:, None, :]
        smin = jnp.min(jnp.where(hit, si, jnp.int32(S)), axis=1)
        y_ref[...] = jnp.where(si == smin[:, None, :], jnp.int32(_NEG),
                               yr).reshape(R, Vp)
        return colmax, smin * _LANES + lane

    banks = []
    for _ in range(_NBANKS):
        yr = y_ref[...].reshape(R, S, _LANES)
        banks.append(extract_bank(yr, colmax_of(yr)))
    cv = jnp.concatenate([b[0] for b in banks], axis=1)
    ci = jnp.concatenate([b[1] for b in banks], axis=1)
    cv, ci = _rank_compact(cv, ci, W)

    def round_body(carry):
        cv, ci, _ = carry
        yr = y_ref[...].reshape(R, S, _LANES)
        colmax = colmax_of(yr)
        bankmax = jnp.max(colmax, axis=1, keepdims=True)
        done = jnp.all(bankmax < cv[:, K - 1:K])

        def merge(args):
            cv, ci = args
            bv, bi = extract_bank(yr, colmax)
            mcv = jnp.concatenate([cv, bv], axis=1)
            mci = jnp.concatenate([ci, bi], axis=1)
            return _rank_compact(mcv, mci, W)

        cv, ci = jax.lax.cond(done, lambda a: a, merge, (cv, ci))
        return cv, ci, done

    cv, ci, _ = jax.lax.while_loop(
        lambda c: jnp.logical_not(c[2]), round_body, (cv, ci, jnp.bool_(False)))

    v = _from_sortable_i32(cv[:, :K])
    m = jnp.max(v, axis=1, keepdims=True)
    e = jnp.exp(v - m)
    probs_ref[...] = e / jnp.sum(e, axis=1, keepdims=True)
    idx_ref[...] = ci[:, :K]


def _topk_probs(logits, K, R, interpret=False):
    B, V = logits.shape
    Vp = ((V + _LANES - 1) // _LANES) * _LANES
    body = functools.partial(_topk_body, V=V, Vp=Vp, R=R, K=K)
    probs, idx = pl.pallas_call(
        body,
        grid=(B // R,),
        in_specs=[pl.BlockSpec((R, Vp), lambda i: (i, 0))],
        out_specs=[
            pl.BlockSpec((R, K), lambda i: (i, 0)),
            pl.BlockSpec((R, K), lambda i: (i, 0)),
        ],
        out_shape=[
            jax.ShapeDtypeStruct((B, K), jnp.float32),
            jax.ShapeDtypeStruct((B, K), jnp.int32),
        ],
        scratch_shapes=[pltpu.VMEM((R, Vp), jnp.int32)],
        interpret=interpret,
    )(logits)
    return probs, idx


@jax.jit
def kernel(logits):
    B, V = logits.shape
    K = min(50, V)
    R = 8 if B % 8 == 0 else 1
    probs, indices = _topk_probs(logits, K, R)
    sample_key = jax.random.key(42)
    sampled = jax.random.categorical(sample_key, jnp.log(probs + 1e-20), axis=-1)
    tokens = jnp.take_along_axis(indices, sampled[:, None], axis=1).squeeze(-1)
    return tokens, probs


# speedup vs baseline: 5.6881x; 2.5101x over previous
import functools

import jax
import jax.numpy as jnp
from jax.experimental import pallas as pl
from jax.experimental.pallas import tpu as pltpu

_LANES = 128
_NEG = -(2**31)
_BIGIDX = 2**31 - 1
_NBANKS = 4


def _to_sortable_i32(x):
    s = jax.lax.bitcast_convert_type(x, jnp.int32)
    return s ^ (jax.lax.shift_right_arithmetic(s, 31) & jnp.int32(0x7FFFFFFF))


def _from_sortable_i32(y):
    s = y ^ (jax.lax.shift_right_arithmetic(y, 31) & jnp.int32(0x7FFFFFFF))
    return jax.lax.bitcast_convert_type(s, jnp.float32)


def _ranks_of(cv, ci):
    R, W = cv.shape
    ranks = jnp.zeros((R, W), jnp.int32)
    CH = 128
    for j0 in range(0, W, CH):
        cvj = jax.lax.broadcast_in_dim(cv[:, j0:j0 + CH], (R, CH, W), (0, 1))
        cij = jax.lax.broadcast_in_dim(ci[:, j0:j0 + CH], (R, CH, W), (0, 1))
        cvi = jax.lax.broadcast_in_dim(cv, (R, CH, W), (0, 2))
        cii = jax.lax.broadcast_in_dim(ci, (R, CH, W), (0, 2))
        better = (cvj > cvi) | ((cvj == cvi) & (cij < cii))
        ranks = ranks + jnp.sum(better.astype(jnp.int32), axis=1)
    return ranks


def _compact(cv, ci, ranks, wout):
    R, W = cv.shape
    slot = jax.lax.broadcasted_iota(jnp.int32, (R, W, wout), 2)
    oh = jax.lax.broadcast_in_dim(ranks, (R, W, wout), (0, 1)) == slot
    cvb = jax.lax.broadcast_in_dim(cv, (R, W, wout), (0, 1))
    cib = jax.lax.broadcast_in_dim(ci, (R, W, wout), (0, 1))
    scv = jnp.sum(jnp.where(oh, cvb, 0), axis=1)
    sci = jnp.sum(jnp.where(oh, cib, 0), axis=1)
    return scv, sci


def _topk_body(x_ref, probs_ref, idx_ref, y_ref, cv_ref, ci_ref, rk_ref, *,
               S, R, K):
    W = _NBANKS * _LANES

    y_ref[...] = _to_sortable_i32(x_ref[...])
    lane = jax.lax.broadcasted_iota(jnp.int32, (R, _LANES), 1)
    si = jax.lax.broadcasted_iota(jnp.int32, (R, S, _LANES), 1)

    def extract_bank(yr, colmax):
        hit = yr == colmax[:, None, :]
        smin = jnp.min(jnp.where(hit, si, jnp.int32(S)), axis=1)
        y_ref[...] = jnp.where(si == smin[:, None, :], jnp.int32(_NEG), yr)
        return colmax, smin * _LANES + lane

    banks = []
    for _ in range(_NBANKS):
        yr = y_ref[...]
        banks.append(extract_bank(yr, jnp.max(yr, axis=1)))
    cv_ref[...] = jnp.concatenate([b[0] for b in banks], axis=1)
    ci_ref[...] = jnp.concatenate([b[1] for b in banks], axis=1)
    rk_ref[...] = _ranks_of(cv_ref[...], ci_ref[...])

    def round_body(_):
        yr = y_ref[...]
        colmax = jnp.max(yr, axis=1)
        bankmax = jnp.max(colmax, axis=1, keepdims=True)
        t50 = jnp.min(
            jnp.where(rk_ref[...] < K, cv_ref[...], jnp.int32(_BIGIDX)),
            axis=1, keepdims=True)
        done = jnp.all(bankmax < t50)

        @pl.when(jnp.logical_not(done))
        def merge():
            bv, bi = extract_bank(yr, colmax)
            mcv = jnp.concatenate([cv_ref[...], bv], axis=1)
            mci = jnp.concatenate([ci_ref[...], bi], axis=1)
            mranks = _ranks_of(mcv, mci)
            ncv, nci = _compact(mcv, mci, mranks, W)
            cv_ref[...] = ncv
            ci_ref[...] = nci
            rk_ref[...] = jax.lax.broadcasted_iota(jnp.int32, (R, W), 1)

        return done

    jax.lax.while_loop(lambda d: jnp.logical_not(d), round_body,
                       jnp.bool_(False))

    sv, sidx = _compact(cv_ref[...], ci_ref[...], rk_ref[...], 64)
    v = _from_sortable_i32(sv[:, :K])
    m = jnp.max(v, axis=1, keepdims=True)
    e = jnp.exp(v - m)
    probs_ref[...] = e / jnp.sum(e, axis=1, keepdims=True)
    idx_ref[...] = sidx[:, :K]


def _topk_probs(logits, K, R, interpret=False):
    B, V = logits.shape
    Vp = ((V + _LANES - 1) // _LANES) * _LANES
    S = Vp // _LANES
    xp = logits
    if Vp != V:
        xp = jnp.pad(logits, ((0, 0), (0, Vp - V)), constant_values=-jnp.inf)
    xp = xp.reshape(B, S, _LANES)
    body = functools.partial(_topk_body, S=S, R=R, K=K)
    probs, idx = pl.pallas_call(
        body,
        grid=(B // R,),
        in_specs=[pl.BlockSpec((R, S, _LANES), lambda i: (i, 0, 0))],
        out_specs=[
            pl.BlockSpec((R, K), lambda i: (i, 0)),
            pl.BlockSpec((R, K), lambda i: (i, 0)),
        ],
        out_shape=[
            jax.ShapeDtypeStruct((B, K), jnp.float32),
            jax.ShapeDtypeStruct((B, K), jnp.int32),
        ],
        scratch_shapes=[
            pltpu.VMEM((R, S, _LANES), jnp.int32),
            pltpu.VMEM((R, _NBANKS * _LANES), jnp.int32),
            pltpu.VMEM((R, _NBANKS * _LANES), jnp.int32),
            pltpu.VMEM((R, _NBANKS * _LANES), jnp.int32),
        ],
        interpret=interpret,
    )(xp)
    return probs, idx


@jax.jit
def kernel(logits):
    B, V = logits.shape
    K = min(50, V)
    R = 8 if B % 8 == 0 else 1
    probs, indices = _topk_probs(logits, K, R)
    sample_key = jax.random.key(42)
    sampled = jax.random.categorical(sample_key, jnp.log(probs + 1e-20), axis=-1)
    tokens = jnp.take_along_axis(indices, sampled[:, None], axis=1).squeeze(-1)
    return tokens, probs
